# baseline (speedup 1.0000x reference)
import functools

import jax
import jax.numpy as jnp
from jax import lax
from jax.experimental import pallas as pl
from jax.experimental.pallas import tpu as pltpu
from jax.experimental.pallas import tpu_sc as plsc

N = 524288
M = N * 40
BBITS = 15
B = 1 << BBITS
SHIFT = 32 - BBITS
NC = 2
NS = 16
NW = NC * NS
L = 16

EPT1 = N // NW
CHUNK = 2048
PPT = M // NW
PCH = 4096
SLC = B // NS

_mesh2 = plsc.VectorSubcoreMesh(core_axis_name="c", subcore_axis_name="s",
                                num_cores=2)
_mesh1 = plsc.VectorSubcoreMesh(core_axis_name="c", subcore_axis_name="s",
                                num_cores=1)


def _wid():
    return lax.axis_index("s") * NC + lax.axis_index("c")


def _bucket(vals_f32):
    bits = lax.bitcast_convert_type(vals_f32, jnp.int32)
    key = jnp.where(bits < 0, jnp.bitwise_not(bits),
                    jnp.bitwise_xor(bits, jnp.int32(-2147483648)))
    return lax.shift_right_logical(key, SHIFT)


@functools.partial(
    pl.kernel,
    out_type=(
        jax.ShapeDtypeStruct((N,), jnp.float32),
        jax.ShapeDtypeStruct((NW * 3 * B,), jnp.int32),
    ),
    mesh=_mesh2,
    compiler_params=pltpu.CompilerParams(needs_layout_passes=False),
    scratch_types=[
        pltpu.VMEM((CHUNK,), jnp.float32),
        pltpu.VMEM((CHUNK,), jnp.float32),
        pltpu.VMEM((CHUNK,), jnp.float32),
        pltpu.VMEM((B,), jnp.int32),
        pltpu.VMEM((B,), jnp.int32),
        pltpu.VMEM((B,), jnp.int32),
    ],
)
def _k1_hist(pred_hbm, gt_hbm, c_hbm, hists_hbm,
             pbuf, gbuf, cbuf, hg, hc, hp):
    wid = _wid()
    zero = jnp.zeros((L,), jnp.int32)

    def _zero(i, _):
        hg[pl.ds(i * L, L)] = zero
        hc[pl.ds(i * L, L)] = zero
        hp[pl.ds(i * L, L)] = zero
        return 0
    lax.fori_loop(0, B // L, _zero, 0)

    ones = jnp.ones((L,), jnp.int32)
    base = wid * EPT1

    def _chunk(k, _):
        off = base + k * CHUNK
        pltpu.sync_copy(pred_hbm.at[pl.ds(off, CHUNK)], pbuf)
        pltpu.sync_copy(gt_hbm.at[pl.ds(off, CHUNK)], gbuf)

        def _vec(i, _):
            p = pbuf[pl.ds(i * L, L)]
            g = gbuf[pl.ds(i * L, L)]
            cv = p - g
            cbuf[pl.ds(i * L, L)] = cv
            plsc.addupdate_scatter(hg, [_bucket(g)], ones)
            plsc.addupdate_scatter(hc, [_bucket(cv)], ones)
            plsc.addupdate_scatter(hp, [_bucket(p)], ones)
            return 0
        lax.fori_loop(0, CHUNK // L, _vec, 0)
        pltpu.sync_copy(cbuf, c_hbm.at[pl.ds(off, CHUNK)])
        return 0
    lax.fori_loop(0, EPT1 // CHUNK, _chunk, 0)

    hb = wid * (3 * B)
    pltpu.sync_copy(hg, hists_hbm.at[pl.ds(hb, B)])
    pltpu.sync_copy(hc, hists_hbm.at[pl.ds(hb + B, B)])
    pltpu.sync_copy(hp, hists_hbm.at[pl.ds(hb + 2 * B, B)])


@functools.partial(
    pl.kernel,
    out_type=jax.ShapeDtypeStruct((3 * B,), jnp.float32),
    mesh=_mesh1,
    compiler_params=pltpu.CompilerParams(needs_layout_passes=False),
    scratch_types=[
        pltpu.VMEM((SLC,), jnp.int32),
        pltpu.VMEM((SLC,), jnp.int32),
        pltpu.VMEM((SLC,), jnp.float32),
        pltpu.VMEM_SHARED((3 * B,), jnp.int32),
    ],
)
def _k2_scan(hists_hbm, mid_hbm, acc, tmp, midb, sh_hist):
    wid = lax.axis_index("s")
    zero = jnp.zeros((L,), jnp.int32)

    def _per_array(arr, _):
        def _zeroacc(i, _):
            acc[pl.ds(i * L, L)] = zero
            return 0
        lax.fori_loop(0, SLC // L, _zeroacc, 0)

        def _merge(t, _):
            pltpu.sync_copy(
                hists_hbm.at[pl.ds(t * (3 * B) + arr * B + wid * SLC, SLC)], tmp)

            def _add(i, _):
                s = pl.ds(i * L, L)
                acc[s] = acc[s] + tmp[s]
                return 0
            lax.fori_loop(0, SLC // L, _add, 0)
            return 0
        lax.fori_loop(0, NW, _merge, 0)
        pltpu.sync_copy(acc, sh_hist.at[pl.ds(arr * B + wid * SLC, SLC)])
        return 0
    lax.fori_loop(0, 3, _per_array, 0)

    plsc.subcore_barrier()

    inv_n = jnp.float32(1.0 / N)

    def _per_array2(arr, _):
        def _pre(t, off):
            pltpu.sync_copy(sh_hist.at[pl.ds(arr * B + t * SLC, SLC)], tmp)

            def _s(i, o):
                return o + jnp.sum(tmp[pl.ds(i * L, L)])
            return lax.fori_loop(0, SLC // L, _s, off)
        off0 = lax.fori_loop(0, wid, _pre, jnp.int32(0))

        pltpu.sync_copy(sh_hist.at[pl.ds(arr * B + wid * SLC, SLC)], tmp)

        def _scan(i, off):
            h = tmp[pl.ds(i * L, L)]
            incl = jnp.cumsum(h)
            excl = (incl - h) + off
            mid = (excl.astype(jnp.float32)
                   + (h.astype(jnp.float32) - 1.0) * 0.5) * inv_n
            midb[pl.ds(i * L, L)] = mid
            return off + jnp.sum(h)
        lax.fori_loop(0, SLC // L, _scan, off0)
        pltpu.sync_copy(midb, mid_hbm.at[pl.ds(arr * B + wid * SLC, SLC)])
        return 0
    lax.fori_loop(0, 3, _per_array2, 0)


@functools.partial(
    pl.kernel,
    out_type=(
        jax.ShapeDtypeStruct((NW * L,), jnp.float32),
        jax.ShapeDtypeStruct((NW * L,), jnp.float32),
        jax.ShapeDtypeStruct((NW * L,), jnp.float32),
    ),
    mesh=_mesh2,
    compiler_params=pltpu.CompilerParams(needs_layout_passes=False),
    scratch_types=[
        pltpu.VMEM((B,), jnp.float32),
        pltpu.VMEM((B,), jnp.float32),
        pltpu.VMEM((B,), jnp.float32),
        pltpu.VMEM((CHUNK,), jnp.float32),
        pltpu.VMEM((CHUNK,), jnp.float32),
        pltpu.VMEM((L,), jnp.float32),
    ],
)
def _k3_dots(pred_hbm, gt_hbm, mid_hbm, pad_hbm, rank_hbm, mse_hbm,
             mg_t, mc_t, mp_t, pbuf, gbuf, ostg):
    wid = _wid()
    pltpu.sync_copy(mid_hbm.at[pl.ds(0, B)], mg_t)
    pltpu.sync_copy(mid_hbm.at[pl.ds(B, B)], mc_t)
    pltpu.sync_copy(mid_hbm.at[pl.ds(2 * B, B)], mp_t)

    base = wid * EPT1
    zf = jnp.zeros((L,), jnp.float32)

    def _chunk(k, accs):
        ap, ar, am = accs
        off = base + k * CHUNK
        pltpu.sync_copy(pred_hbm.at[pl.ds(off, CHUNK)], pbuf)
        pltpu.sync_copy(gt_hbm.at[pl.ds(off, CHUNK)], gbuf)

        def _vec(i, accs2):
            ap2, ar2, am2 = accs2
            p = pbuf[pl.ds(i * L, L)]
            g = gbuf[pl.ds(i * L, L)]
            cv = p - g
            mg = plsc.load_gather(mg_t, [_bucket(g)])
            mc = plsc.load_gather(mc_t, [_bucket(cv)])
            mp = plsc.load_gather(mp_t, [_bucket(p)])
            dpad = mc - mg
            drank = mp - mg
            return (ap2 + dpad * dpad, ar2 + drank * drank, am2 + cv * cv)
        return lax.fori_loop(0, CHUNK // L, _vec, (ap, ar, am))
    ap, ar, am = lax.fori_loop(0, EPT1 // CHUNK, _chunk, (zf, zf, zf))

    ostg[pl.ds(0, L)] = ap
    pltpu.sync_copy(ostg, pad_hbm.at[pl.ds(wid * L, L)])
    ostg[pl.ds(0, L)] = ar
    pltpu.sync_copy(ostg, rank_hbm.at[pl.ds(wid * L, L)])
    ostg[pl.ds(0, L)] = am
    pltpu.sync_copy(ostg, mse_hbm.at[pl.ds(wid * L, L)])


NCHUNK = PPT // PCH


@functools.partial(
    pl.kernel,
    out_type=jax.ShapeDtypeStruct((NW * L,), jnp.float32),
    mesh=_mesh2,
    compiler_params=pltpu.CompilerParams(needs_layout_passes=False),
    scratch_types=[
        pltpu.VMEM_SHARED((N // 2,), jnp.float32),
        pltpu.VMEM((PCH,), jnp.int32),
        pltpu.VMEM((PCH,), jnp.int32),
        pltpu.VMEM((PCH,), jnp.int32),
        pltpu.VMEM((PCH,), jnp.int32),
        pltpu.VMEM((PCH,), jnp.int32),
        pltpu.VMEM((PCH,), jnp.int32),
        pltpu.VMEM((PCH,), jnp.int32),
        pltpu.VMEM((PCH,), jnp.int32),
        pltpu.VMEM((PCH,), jnp.float32),
        pltpu.VMEM((PCH,), jnp.float32),
        pltpu.VMEM((PCH,), jnp.float32),
        pltpu.VMEM((PCH,), jnp.float32),
        pltpu.VMEM((PCH,), jnp.float32),
        pltpu.VMEM((PCH,), jnp.float32),
        pltpu.VMEM((PCH,), jnp.float32),
        pltpu.VMEM((L,), jnp.float32),
        pltpu.SemaphoreType.DMA,
        pltpu.SemaphoreType.DMA,
        pltpu.SemaphoreType.DMA,
        pltpu.SemaphoreType.DMA,
        pltpu.SemaphoreType.DMA,
        pltpu.SemaphoreType.DMA,
        pltpu.SemaphoreType.DMA,
        pltpu.SemaphoreType.DMA,
    ],
)
def _k4_pairs(c_hbm, a_hbm, b_hbm, age_hbm,
              sh_cp, a0, a1, b0, b1, ma0, ma1, mb0, mb1,
              ca0, ca1, cb0, cb1, clo, chi, pkb, ostg,
              ia0, ia1, ib0, ib1, ga0, ga1, gb0, gb1):
    sid = lax.axis_index("s")
    wid = _wid()
    half = N // 2
    wseg = half // NS

    def _stage(j, _):
        woff = sid * wseg + j * PCH
        pltpu.sync_copy(c_hbm.at[pl.ds(woff, PCH)], clo)
        pltpu.sync_copy(c_hbm.at[pl.ds(woff + half, PCH)], chi)

        def _pk(i, _):
            s = pl.ds(i * L, L)
            blo = lax.bitcast_convert_type(clo[s], jnp.int32)
            bhi = lax.bitcast_convert_type(chi[s], jnp.int32)
            rlo = lax.shift_right_logical(
                blo + 0x7FFF + (lax.shift_right_logical(blo, 16) & 1), 16)
            rhi = lax.shift_right_logical(
                bhi + 0x7FFF + (lax.shift_right_logical(bhi, 16) & 1), 16)
            pkb[s] = lax.bitcast_convert_type(rlo | lax.shift_left(rhi, 16),
                                              jnp.float32)
            return 0
        lax.fori_loop(0, PCH // L, _pk, 0, unroll=8)
        pltpu.sync_copy(pkb, sh_cp.at[pl.ds(woff, PCH)])
        return 0
    lax.fori_loop(0, wseg // PCH, _stage, 0)
    plsc.subcore_barrier()

    base = wid * PPT
    zf = jnp.zeros((L,), jnp.float32)
    abufs, bbufs = (a0, a1), (b0, b1)
    mabufs, mbbufs = (ma0, ma1), (mb0, mb1)
    cabufs, cbbufs = (ca0, ca1), (cb0, cb1)
    iasems, ibsems = (ia0, ia1), (ib0, ib1)
    gasems, gbsems = (ga0, ga1), (gb0, gb1)
    wmask = jnp.int32(half - 1)

    def _issue_idx(k, s):
        kk = lax.rem(k, jnp.int32(NCHUNK))
        off = base + kk * PCH
        pltpu.async_copy(a_hbm.at[pl.ds(off, PCH)], abufs[s], iasems[s])
        pltpu.async_copy(b_hbm.at[pl.ds(off, PCH)], bbufs[s], ibsems[s])

    def _wait_idx(s):
        pltpu.make_async_copy(a_hbm.at[pl.ds(0, PCH)], abufs[s], iasems[s]).wait()
        pltpu.make_async_copy(b_hbm.at[pl.ds(0, PCH)], bbufs[s], ibsems[s]).wait()

    def _mask_idx(s):
        def _m(i, _):
            sl = pl.ds(i * L, L)
            mabufs[s][sl] = abufs[s][sl] & wmask
            mbbufs[s][sl] = bbufs[s][sl] & wmask
            return 0
        lax.fori_loop(0, PCH // L, _m, 0, unroll=8)

    def _b_from_hbm(k):
        return lax.rem(k, jnp.int32(16)) < 9

    def _issue_gather(k, s):
        pltpu.async_copy(sh_cp.at[mabufs[s]], cabufs[s], gasems[s])
        hbm = _b_from_hbm(k)

        @pl.when(hbm)
        def _():
            pltpu.async_copy(c_hbm.at[bbufs[s]], cbbufs[s], gbsems[s])

        @pl.when(jnp.logical_not(hbm))
        def _():
            pltpu.async_copy(sh_cp.at[mbbufs[s]], cbbufs[s], gbsems[s])

    def _wait_gather(s):
        pltpu.make_async_copy(sh_cp.at[mabufs[s]], cabufs[s], gasems[s]).wait()
        pltpu.make_async_copy(sh_cp.at[mbbufs[s]], cbbufs[s], gbsems[s]).wait()

    def _decode(wordf, idx):
        word = lax.bitcast_convert_type(wordf, jnp.int32)
        hi = lax.shift_right_logical(idx, 18) > 0
        bits = jnp.where(hi, word & jnp.int32(-65536), lax.shift_left(word, 16))
        return lax.bitcast_convert_type(bits, jnp.float32)

    def _compute(k, s, acc):
        bhbm = _b_from_hbm(k)

        def _vec(i, acc2):
            sl = pl.ds(i * L, L)
            va = _decode(cabufs[s][sl], abufs[s][sl])
            braw = cbbufs[s][sl]
            vb = jnp.where(bhbm, braw, _decode(braw, bbufs[s][sl]))
            d = va - vb
            return acc2 + d * d
        return lax.fori_loop(0, PCH // L, _vec, acc, unroll=8)

    _issue_idx(jnp.int32(0), 0)
    _issue_idx(jnp.int32(1), 1)

    def _super(h, acc):
        k0 = h * 2
        _wait_idx(0)
        _mask_idx(0)
        _issue_gather(k0, 0)
        _wait_idx(1)
        _mask_idx(1)
        _issue_gather(k0 + 1, 1)
        _wait_gather(0)
        _issue_idx(k0 + 2, 0)
        acc = _compute(k0, 0, acc)
        _wait_gather(1)
        _issue_idx(k0 + 3, 1)
        acc = _compute(k0 + 1, 1, acc)
        return acc
    acc = lax.fori_loop(0, NCHUNK // 2, _super, zf)
    _wait_idx(0)
    _wait_idx(1)

    ostg[pl.ds(0, L)] = acc
    pltpu.sync_copy(ostg, age_hbm.at[pl.ds(wid * L, L)])


def kernel(mem_pred, mem_gt, a, b):
    c, hists = _k1_hist(mem_pred, mem_gt)
    mid = _k2_scan(hists)
    pad_p, rank_p, mse_p = _k3_dots(mem_pred, mem_gt, mid)
    age_p = _k4_pairs(c, a, b)
    inv_n = jnp.float32(1.0 / N)
    l_pad = jnp.sum(pad_p) * inv_n
    l_rank = jnp.sum(rank_p) * inv_n
    mse = jnp.sum(mse_p) * inv_n
    l_age = jnp.sum(age_p) * jnp.float32(1.0 / M)
    return 20.0 * (l_pad + l_rank + l_age) + mse


# speedup vs baseline: 1.3844x; 1.3844x over previous
import functools

import jax
import jax.numpy as jnp
from jax import lax
from jax.experimental import pallas as pl
from jax.experimental.pallas import tpu as pltpu
from jax.experimental.pallas import tpu_sc as plsc

N = 524288
M = N * 40
BBITS = 15
B = 1 << BBITS
SHIFT = 32 - BBITS
NC = 2
NS = 16
NW = NC * NS
L = 16

EPT1 = N // NW
CHUNK = 2048
PPT = M // NW
PCH = 4096
SLC = B // NS

_mesh2 = plsc.VectorSubcoreMesh(core_axis_name="c", subcore_axis_name="s",
                                num_cores=2)
_mesh1 = plsc.VectorSubcoreMesh(core_axis_name="c", subcore_axis_name="s",
                                num_cores=1)


def _wid():
    return lax.axis_index("s") * NC + lax.axis_index("c")


def _bucket(vals_f32):
    bits = lax.bitcast_convert_type(vals_f32, jnp.int32)
    key = jnp.where(bits < 0, jnp.bitwise_not(bits),
                    jnp.bitwise_xor(bits, jnp.int32(-2147483648)))
    return lax.shift_right_logical(key, SHIFT)


@functools.partial(
    pl.kernel,
    out_type=(
        jax.ShapeDtypeStruct((N,), jnp.float32),
        jax.ShapeDtypeStruct((NW * 3 * B,), jnp.int32),
    ),
    mesh=_mesh2,
    compiler_params=pltpu.CompilerParams(needs_layout_passes=False),
    scratch_types=[
        pltpu.VMEM((CHUNK,), jnp.float32),
        pltpu.VMEM((CHUNK,), jnp.float32),
        pltpu.VMEM((CHUNK,), jnp.float32),
        pltpu.VMEM((B,), jnp.int32),
        pltpu.VMEM((B,), jnp.int32),
        pltpu.VMEM((B,), jnp.int32),
    ],
)
def _k1_hist(pred_hbm, gt_hbm, c_hbm, hists_hbm,
             pbuf, gbuf, cbuf, hg, hc, hp):
    wid = _wid()
    zero = jnp.zeros((L,), jnp.int32)

    def _zero(i, _):
        hg[pl.ds(i * L, L)] = zero
        hc[pl.ds(i * L, L)] = zero
        hp[pl.ds(i * L, L)] = zero
        return 0
    lax.fori_loop(0, B // L, _zero, 0, unroll=8)

    ones = jnp.ones((L,), jnp.int32)
    base = wid * EPT1

    def _chunk(k, _):
        off = base + k * CHUNK
        pltpu.sync_copy(pred_hbm.at[pl.ds(off, CHUNK)], pbuf)
        pltpu.sync_copy(gt_hbm.at[pl.ds(off, CHUNK)], gbuf)

        def _vec(i, _):
            p = pbuf[pl.ds(i * L, L)]
            g = gbuf[pl.ds(i * L, L)]
            cv = p - g
            cbuf[pl.ds(i * L, L)] = cv
            plsc.addupdate_scatter(hg, [_bucket(g)], ones)
            plsc.addupdate_scatter(hc, [_bucket(cv)], ones)
            plsc.addupdate_scatter(hp, [_bucket(p)], ones)
            return 0
        lax.fori_loop(0, CHUNK // L, _vec, 0, unroll=4)
        pltpu.sync_copy(cbuf, c_hbm.at[pl.ds(off, CHUNK)])
        return 0
    lax.fori_loop(0, EPT1 // CHUNK, _chunk, 0)

    hb = wid * (3 * B)
    pltpu.sync_copy(hg, hists_hbm.at[pl.ds(hb, B)])
    pltpu.sync_copy(hc, hists_hbm.at[pl.ds(hb + B, B)])
    pltpu.sync_copy(hp, hists_hbm.at[pl.ds(hb + 2 * B, B)])


@functools.partial(
    pl.kernel,
    out_type=jax.ShapeDtypeStruct((3 * B,), jnp.float32),
    mesh=_mesh1,
    compiler_params=pltpu.CompilerParams(needs_layout_passes=False),
    scratch_types=[
        pltpu.VMEM((NW * SLC,), jnp.int32),
        pltpu.VMEM((3 * SLC,), jnp.int32),
        pltpu.VMEM((SLC,), jnp.float32),
        pltpu.VMEM((L,), jnp.int32),
        pltpu.VMEM((NS * L,), jnp.int32),
        pltpu.VMEM_SHARED((NS * L,), jnp.int32),
        pltpu.SemaphoreType.DMA,
    ],
)
def _k2_scan(hists_hbm, mid_hbm, fan, acc3, midb, trow, tall, sh_tot, sem):
    wid = lax.axis_index("s")
    lanes = lax.iota(jnp.int32, L)

    def _per_array(arr, tots):
        def _fire(t, _):
            pltpu.async_copy(
                hists_hbm.at[pl.ds(t * (3 * B) + arr * B + wid * SLC, SLC)],
                fan.at[pl.ds(t * SLC, SLC)], sem)
            return 0
        lax.fori_loop(0, NW, _fire, 0)

        def _drain(t, _):
            pltpu.make_async_copy(
                hists_hbm.at[pl.ds(0, SLC)], fan.at[pl.ds(0, SLC)], sem).wait()
            return 0
        lax.fori_loop(0, NW, _drain, 0)

        def _add(i, tot):
            s = pl.ds(arr * SLC + i * L, L)

            def _srcsum(t, v):
                return v + fan[pl.ds(t * SLC + i * L, L)]
            v = lax.fori_loop(0, NW, _srcsum, jnp.zeros((L,), jnp.int32),
                              unroll=8)
            acc3[s] = v
            return tot + jnp.sum(v)
        tot = lax.fori_loop(0, SLC // L, _add, jnp.int32(0), unroll=2)
        return jnp.where(lanes == arr, tot, tots)
    tots = lax.fori_loop(0, 3, _per_array, jnp.zeros((L,), jnp.int32))

    trow[pl.ds(0, L)] = tots
    pltpu.sync_copy(trow, sh_tot.at[pl.ds(wid * L, L)])
    plsc.subcore_barrier()
    pltpu.sync_copy(sh_tot, tall)

    inv_n = jnp.float32(1.0 / N)

    def _per_array2(arr, _):
        tvec = plsc.load_gather(tall, [lanes * L + arr])
        off0 = jnp.sum(jnp.where(lanes < wid, tvec, 0))

        def _scan(i, off):
            h = acc3[pl.ds(arr * SLC + i * L, L)]
            incl = jnp.cumsum(h)
            excl = (incl - h) + off
            mid = (excl.astype(jnp.float32)
                   + (h.astype(jnp.float32) - 1.0) * 0.5) * inv_n
            midb[pl.ds(i * L, L)] = mid
            return off + jnp.sum(h)
        lax.fori_loop(0, SLC // L, _scan, off0)
        pltpu.sync_copy(midb, mid_hbm.at[pl.ds(arr * B + wid * SLC, SLC)])
        return 0
    lax.fori_loop(0, 3, _per_array2, 0)


@functools.partial(
    pl.kernel,
    out_type=(
        jax.ShapeDtypeStruct((NW * L,), jnp.float32),
        jax.ShapeDtypeStruct((NW * L,), jnp.float32),
        jax.ShapeDtypeStruct((NW * L,), jnp.float32),
    ),
    mesh=_mesh2,
    compiler_params=pltpu.CompilerParams(needs_layout_passes=False),
    scratch_types=[
        pltpu.VMEM((B,), jnp.float32),
        pltpu.VMEM((B,), jnp.float32),
        pltpu.VMEM((B,), jnp.float32),
        pltpu.VMEM((CHUNK,), jnp.float32),
        pltpu.VMEM((CHUNK,), jnp.float32),
        pltpu.VMEM((L,), jnp.float32),
    ],
)
def _k3_dots(pred_hbm, gt_hbm, mid_hbm, pad_hbm, rank_hbm, mse_hbm,
             mg_t, mc_t, mp_t, pbuf, gbuf, ostg):
    wid = _wid()
    pltpu.sync_copy(mid_hbm.at[pl.ds(0, B)], mg_t)
    pltpu.sync_copy(mid_hbm.at[pl.ds(B, B)], mc_t)
    pltpu.sync_copy(mid_hbm.at[pl.ds(2 * B, B)], mp_t)

    base = wid * EPT1
    zf = jnp.zeros((L,), jnp.float32)

    def _chunk(k, accs):
        ap, ar, am = accs
        off = base + k * CHUNK
        pltpu.sync_copy(pred_hbm.at[pl.ds(off, CHUNK)], pbuf)
        pltpu.sync_copy(gt_hbm.at[pl.ds(off, CHUNK)], gbuf)

        def _vec(i, accs2):
            ap2, ar2, am2 = accs2
            p = pbuf[pl.ds(i * L, L)]
            g = gbuf[pl.ds(i * L, L)]
            cv = p - g
            mg = plsc.load_gather(mg_t, [_bucket(g)])
            mc = plsc.load_gather(mc_t, [_bucket(cv)])
            mp = plsc.load_gather(mp_t, [_bucket(p)])
            dpad = mc - mg
            drank = mp - mg
            return (ap2 + dpad * dpad, ar2 + drank * drank, am2 + cv * cv)
        return lax.fori_loop(0, CHUNK // L, _vec, (ap, ar, am), unroll=4)
    ap, ar, am = lax.fori_loop(0, EPT1 // CHUNK, _chunk, (zf, zf, zf))

    ostg[pl.ds(0, L)] = ap
    pltpu.sync_copy(ostg, pad_hbm.at[pl.ds(wid * L, L)])
    ostg[pl.ds(0, L)] = ar
    pltpu.sync_copy(ostg, rank_hbm.at[pl.ds(wid * L, L)])
    ostg[pl.ds(0, L)] = am
    pltpu.sync_copy(ostg, mse_hbm.at[pl.ds(wid * L, L)])


NCHUNK = PPT // PCH


@functools.partial(
    pl.kernel,
    out_type=jax.ShapeDtypeStruct((NW * L,), jnp.float32),
    mesh=_mesh2,
    compiler_params=pltpu.CompilerParams(needs_layout_passes=False),
    scratch_types=[
        pltpu.VMEM_SHARED((N // 2,), jnp.float32),
        pltpu.VMEM((PCH,), jnp.int32),
        pltpu.VMEM((PCH,), jnp.int32),
        pltpu.VMEM((PCH,), jnp.int32),
        pltpu.VMEM((PCH,), jnp.int32),
        pltpu.VMEM((PCH,), jnp.int32),
        pltpu.VMEM((PCH,), jnp.int32),
        pltpu.VMEM((PCH,), jnp.int32),
        pltpu.VMEM((PCH,), jnp.int32),
        pltpu.VMEM((PCH,), jnp.float32),
        pltpu.VMEM((PCH,), jnp.float32),
        pltpu.VMEM((PCH,), jnp.float32),
        pltpu.VMEM((PCH,), jnp.float32),
        pltpu.VMEM((PCH,), jnp.float32),
        pltpu.VMEM((PCH,), jnp.float32),
        pltpu.VMEM((PCH,), jnp.float32),
        pltpu.VMEM((L,), jnp.float32),
        pltpu.SemaphoreType.DMA,
        pltpu.SemaphoreType.DMA,
        pltpu.SemaphoreType.DMA,
        pltpu.SemaphoreType.DMA,
        pltpu.SemaphoreType.DMA,
        pltpu.SemaphoreType.DMA,
        pltpu.SemaphoreType.DMA,
        pltpu.SemaphoreType.DMA,
    ],
)
def _k4_pairs(c_hbm, a_hbm, b_hbm, age_hbm,
              sh_cp, a0, a1, b0, b1, ma0, ma1, mb0, mb1,
              ca0, ca1, cb0, cb1, clo, chi, pkb, ostg,
              ia0, ia1, ib0, ib1, ga0, ga1, gb0, gb1):
    sid = lax.axis_index("s")
    wid = _wid()
    half = N // 2
    wseg = half // NS

    def _stage(j, _):
        woff = sid * wseg + j * PCH
        pltpu.sync_copy(c_hbm.at[pl.ds(woff, PCH)], clo)
        pltpu.sync_copy(c_hbm.at[pl.ds(woff + half, PCH)], chi)

        def _pk(i, _):
            s = pl.ds(i * L, L)
            blo = lax.bitcast_convert_type(clo[s], jnp.int32)
            bhi = lax.bitcast_convert_type(chi[s], jnp.int32)
            rlo = lax.shift_right_logical(
                blo + 0x7FFF + (lax.shift_right_logical(blo, 16) & 1), 16)
            rhi = lax.shift_right_logical(
                bhi + 0x7FFF + (lax.shift_right_logical(bhi, 16) & 1), 16)
            pkb[s] = lax.bitcast_convert_type(rlo | lax.shift_left(rhi, 16),
                                              jnp.float32)
            return 0
        lax.fori_loop(0, PCH // L, _pk, 0, unroll=8)
        pltpu.sync_copy(pkb, sh_cp.at[pl.ds(woff, PCH)])
        return 0
    lax.fori_loop(0, wseg // PCH, _stage, 0)
    plsc.subcore_barrier()

    base = wid * PPT
    zf = jnp.zeros((L,), jnp.float32)
    abufs, bbufs = (a0, a1), (b0, b1)
    mabufs, mbbufs = (ma0, ma1), (mb0, mb1)
    cabufs, cbbufs = (ca0, ca1), (cb0, cb1)
    iasems, ibsems = (ia0, ia1), (ib0, ib1)
    gasems, gbsems = (ga0, ga1), (gb0, gb1)
    wmask = jnp.int32(half - 1)

    def _issue_idx(k, s):
        kk = lax.rem(k, jnp.int32(NCHUNK))
        off = base + kk * PCH
        pltpu.async_copy(a_hbm.at[pl.ds(off, PCH)], abufs[s], iasems[s])
        pltpu.async_copy(b_hbm.at[pl.ds(off, PCH)], bbufs[s], ibsems[s])

    def _wait_idx(s):
        pltpu.make_async_copy(a_hbm.at[pl.ds(0, PCH)], abufs[s], iasems[s]).wait()
        pltpu.make_async_copy(b_hbm.at[pl.ds(0, PCH)], bbufs[s], ibsems[s]).wait()

    def _mask_idx(s):
        def _m(i, _):
            sl = pl.ds(i * L, L)
            mabufs[s][sl] = abufs[s][sl] & wmask
            mbbufs[s][sl] = bbufs[s][sl] & wmask
            return 0
        lax.fori_loop(0, PCH // L, _m, 0, unroll=8)

    def _issue_gather(k, s):
        pltpu.async_copy(sh_cp.at[mabufs[s]], cabufs[s], gasems[s])
        pltpu.async_copy(sh_cp.at[mbbufs[s]], cbbufs[s], gbsems[s])

    def _wait_gather(s):
        pltpu.make_async_copy(sh_cp.at[mabufs[s]], cabufs[s], gasems[s]).wait()
        pltpu.make_async_copy(sh_cp.at[mbbufs[s]], cbbufs[s], gbsems[s]).wait()

    def _decode(wordf, idx):
        word = lax.bitcast_convert_type(wordf, jnp.int32)
        hi = lax.shift_right_logical(idx, 18) > 0
        bits = jnp.where(hi, word & jnp.int32(-65536), lax.shift_left(word, 16))
        return lax.bitcast_convert_type(bits, jnp.float32)

    def _compute(k, s, acc):
        del k

        def _vec(i, acc2):
            sl = pl.ds(i * L, L)
            va = _decode(cabufs[s][sl], abufs[s][sl])
            vb = _decode(cbbufs[s][sl], bbufs[s][sl])
            d = va - vb
            return acc2 + d * d
        return lax.fori_loop(0, PCH // L, _vec, acc, unroll=8)

    _issue_idx(jnp.int32(0), 0)
    _issue_idx(jnp.int32(1), 1)

    def _super(h, acc):
        k0 = h * 2
        _wait_idx(0)
        _mask_idx(0)
        _issue_gather(k0, 0)
        _wait_idx(1)
        _mask_idx(1)
        _issue_gather(k0 + 1, 1)
        _wait_gather(0)
        _issue_idx(k0 + 2, 0)
        acc = _compute(k0, 0, acc)
        _wait_gather(1)
        _issue_idx(k0 + 3, 1)
        acc = _compute(k0 + 1, 1, acc)
        return acc
    acc = lax.fori_loop(0, NCHUNK // 2, _super, zf)
    _wait_idx(0)
    _wait_idx(1)

    ostg[pl.ds(0, L)] = acc
    pltpu.sync_copy(ostg, age_hbm.at[pl.ds(wid * L, L)])


def kernel(mem_pred, mem_gt, a, b):
    c, hists = _k1_hist(mem_pred, mem_gt)
    mid = _k2_scan(hists)
    pad_p, rank_p, mse_p = _k3_dots(mem_pred, mem_gt, mid)
    age_p = _k4_pairs(c, a, b)
    inv_n = jnp.float32(1.0 / N)
    l_pad = jnp.sum(pad_p) * inv_n
    l_rank = jnp.sum(rank_p) * inv_n
    mse = jnp.sum(mse_p) * inv_n
    l_age = jnp.sum(age_p) * jnp.float32(1.0 / M)
    return 20.0 * (l_pad + l_rank + l_age) + mse


# speedup vs baseline: 1.3966x; 1.0089x over previous
import functools

import jax
import jax.numpy as jnp
from jax import lax
from jax.experimental import pallas as pl
from jax.experimental.pallas import tpu as pltpu
from jax.experimental.pallas import tpu_sc as plsc

N = 524288
M = N * 40
BBITS = 15
B = 1 << BBITS
SHIFT = 32 - BBITS
NC = 2
NS = 16
NW = NC * NS
L = 16

EPT1 = N // NW
CHUNK = 2048
PPT = M // NW
PCH = 8192
SLC = B // NS

_mesh2 = plsc.VectorSubcoreMesh(core_axis_name="c", subcore_axis_name="s",
                                num_cores=2)
_mesh1 = plsc.VectorSubcoreMesh(core_axis_name="c", subcore_axis_name="s",
                                num_cores=1)


def _wid():
    return lax.axis_index("s") * NC + lax.axis_index("c")


def _bucket(vals_f32):
    bits = lax.bitcast_convert_type(vals_f32, jnp.int32)
    key = jnp.where(bits < 0, jnp.bitwise_not(bits),
                    jnp.bitwise_xor(bits, jnp.int32(-2147483648)))
    return lax.shift_right_logical(key, SHIFT)


@functools.partial(
    pl.kernel,
    out_type=(
        jax.ShapeDtypeStruct((N,), jnp.float32),
        jax.ShapeDtypeStruct((NW * 3 * B,), jnp.int32),
    ),
    mesh=_mesh2,
    compiler_params=pltpu.CompilerParams(needs_layout_passes=False),
    scratch_types=[
        pltpu.VMEM((CHUNK,), jnp.float32),
        pltpu.VMEM((CHUNK,), jnp.float32),
        pltpu.VMEM((CHUNK,), jnp.float32),
        pltpu.VMEM((B,), jnp.int32),
        pltpu.VMEM((B,), jnp.int32),
        pltpu.VMEM((B,), jnp.int32),
    ],
)
def _k1_hist(pred_hbm, gt_hbm, c_hbm, hists_hbm,
             pbuf, gbuf, cbuf, hg, hc, hp):
    wid = _wid()
    zero = jnp.zeros((L,), jnp.int32)

    def _zero(i, _):
        hg[pl.ds(i * L, L)] = zero
        hc[pl.ds(i * L, L)] = zero
        hp[pl.ds(i * L, L)] = zero
        return 0
    lax.fori_loop(0, B // L, _zero, 0, unroll=8)

    ones = jnp.ones((L,), jnp.int32)
    base = wid * EPT1

    def _chunk(k, _):
        off = base + k * CHUNK
        pltpu.sync_copy(pred_hbm.at[pl.ds(off, CHUNK)], pbuf)
        pltpu.sync_copy(gt_hbm.at[pl.ds(off, CHUNK)], gbuf)

        def _vec(i, _):
            p = pbuf[pl.ds(i * L, L)]
            g = gbuf[pl.ds(i * L, L)]
            cv = p - g
            cbuf[pl.ds(i * L, L)] = cv
            plsc.addupdate_scatter(hg, [_bucket(g)], ones)
            plsc.addupdate_scatter(hc, [_bucket(cv)], ones)
            plsc.addupdate_scatter(hp, [_bucket(p)], ones)
            return 0
        lax.fori_loop(0, CHUNK // L, _vec, 0, unroll=4)
        pltpu.sync_copy(cbuf, c_hbm.at[pl.ds(off, CHUNK)])
        return 0
    lax.fori_loop(0, EPT1 // CHUNK, _chunk, 0)

    hb = wid * (3 * B)
    pltpu.sync_copy(hg, hists_hbm.at[pl.ds(hb, B)])
    pltpu.sync_copy(hc, hists_hbm.at[pl.ds(hb + B, B)])
    pltpu.sync_copy(hp, hists_hbm.at[pl.ds(hb + 2 * B, B)])


@functools.partial(
    pl.kernel,
    out_type=jax.ShapeDtypeStruct((3 * B,), jnp.float32),
    mesh=_mesh1,
    compiler_params=pltpu.CompilerParams(needs_layout_passes=False),
    scratch_types=[
        pltpu.VMEM((NW * SLC,), jnp.int32),
        pltpu.VMEM((3 * SLC,), jnp.int32),
        pltpu.VMEM((SLC,), jnp.float32),
        pltpu.VMEM((L,), jnp.int32),
        pltpu.VMEM((NS * L,), jnp.int32),
        pltpu.VMEM_SHARED((NS * L,), jnp.int32),
        pltpu.SemaphoreType.DMA,
    ],
)
def _k2_scan(hists_hbm, mid_hbm, fan, acc3, midb, trow, tall, sh_tot, sem):
    wid = lax.axis_index("s")
    lanes = lax.iota(jnp.int32, L)

    def _per_array(arr, tots):
        def _fire(t, _):
            pltpu.async_copy(
                hists_hbm.at[pl.ds(t * (3 * B) + arr * B + wid * SLC, SLC)],
                fan.at[pl.ds(t * SLC, SLC)], sem)
            return 0
        lax.fori_loop(0, NW, _fire, 0)

        def _drain(t, _):
            pltpu.make_async_copy(
                hists_hbm.at[pl.ds(0, SLC)], fan.at[pl.ds(0, SLC)], sem).wait()
            return 0
        lax.fori_loop(0, NW, _drain, 0)

        def _add(i, tot):
            s = pl.ds(arr * SLC + i * L, L)

            def _srcsum(t, v):
                return v + fan[pl.ds(t * SLC + i * L, L)]
            v = lax.fori_loop(0, NW, _srcsum, jnp.zeros((L,), jnp.int32),
                              unroll=8)
            acc3[s] = v
            return tot + jnp.sum(v)
        tot = lax.fori_loop(0, SLC // L, _add, jnp.int32(0), unroll=2)
        return jnp.where(lanes == arr, tot, tots)
    tots = lax.fori_loop(0, 3, _per_array, jnp.zeros((L,), jnp.int32))

    trow[pl.ds(0, L)] = tots
    pltpu.sync_copy(trow, sh_tot.at[pl.ds(wid * L, L)])
    plsc.subcore_barrier()
    pltpu.sync_copy(sh_tot, tall)

    inv_n = jnp.float32(1.0 / N)

    def _per_array2(arr, _):
        tvec = plsc.load_gather(tall, [lanes * L + arr])
        off0 = jnp.sum(jnp.where(lanes < wid, tvec, 0))

        def _scan(i, off):
            h = acc3[pl.ds(arr * SLC + i * L, L)]
            incl = jnp.cumsum(h)
            excl = (incl - h) + off
            mid = (excl.astype(jnp.float32)
                   + (h.astype(jnp.float32) - 1.0) * 0.5) * inv_n
            midb[pl.ds(i * L, L)] = mid
            return off + jnp.sum(h)
        lax.fori_loop(0, SLC // L, _scan, off0)
        pltpu.sync_copy(midb, mid_hbm.at[pl.ds(arr * B + wid * SLC, SLC)])
        return 0
    lax.fori_loop(0, 3, _per_array2, 0)


@functools.partial(
    pl.kernel,
    out_type=(
        jax.ShapeDtypeStruct((NW * L,), jnp.float32),
        jax.ShapeDtypeStruct((NW * L,), jnp.float32),
        jax.ShapeDtypeStruct((NW * L,), jnp.float32),
    ),
    mesh=_mesh2,
    compiler_params=pltpu.CompilerParams(needs_layout_passes=False),
    scratch_types=[
        pltpu.VMEM((B,), jnp.float32),
        pltpu.VMEM((B,), jnp.float32),
        pltpu.VMEM((B,), jnp.float32),
        pltpu.VMEM((CHUNK,), jnp.float32),
        pltpu.VMEM((CHUNK,), jnp.float32),
        pltpu.VMEM((L,), jnp.float32),
    ],
)
def _k3_dots(pred_hbm, gt_hbm, mid_hbm, pad_hbm, rank_hbm, mse_hbm,
             mg_t, mc_t, mp_t, pbuf, gbuf, ostg):
    wid = _wid()
    pltpu.sync_copy(mid_hbm.at[pl.ds(0, B)], mg_t)
    pltpu.sync_copy(mid_hbm.at[pl.ds(B, B)], mc_t)
    pltpu.sync_copy(mid_hbm.at[pl.ds(2 * B, B)], mp_t)

    base = wid * EPT1
    zf = jnp.zeros((L,), jnp.float32)

    def _chunk(k, accs):
        ap, ar, am = accs
        off = base + k * CHUNK
        pltpu.sync_copy(pred_hbm.at[pl.ds(off, CHUNK)], pbuf)
        pltpu.sync_copy(gt_hbm.at[pl.ds(off, CHUNK)], gbuf)

        def _vec(i, accs2):
            ap2, ar2, am2 = accs2
            p = pbuf[pl.ds(i * L, L)]
            g = gbuf[pl.ds(i * L, L)]
            cv = p - g
            mg = plsc.load_gather(mg_t, [_bucket(g)])
            mc = plsc.load_gather(mc_t, [_bucket(cv)])
            mp = plsc.load_gather(mp_t, [_bucket(p)])
            dpad = mc - mg
            drank = mp - mg
            return (ap2 + dpad * dpad, ar2 + drank * drank, am2 + cv * cv)
        return lax.fori_loop(0, CHUNK // L, _vec, (ap, ar, am), unroll=4)
    ap, ar, am = lax.fori_loop(0, EPT1 // CHUNK, _chunk, (zf, zf, zf))

    ostg[pl.ds(0, L)] = ap
    pltpu.sync_copy(ostg, pad_hbm.at[pl.ds(wid * L, L)])
    ostg[pl.ds(0, L)] = ar
    pltpu.sync_copy(ostg, rank_hbm.at[pl.ds(wid * L, L)])
    ostg[pl.ds(0, L)] = am
    pltpu.sync_copy(ostg, mse_hbm.at[pl.ds(wid * L, L)])


NCHUNK = PPT // PCH


@functools.partial(
    pl.kernel,
    out_type=jax.ShapeDtypeStruct((NW * L,), jnp.float32),
    mesh=_mesh2,
    compiler_params=pltpu.CompilerParams(needs_layout_passes=False),
    scratch_types=[
        pltpu.VMEM_SHARED((N // 2,), jnp.float32),
        pltpu.VMEM((PCH,), jnp.int32),
        pltpu.VMEM((PCH,), jnp.int32),
        pltpu.VMEM((PCH,), jnp.int32),
        pltpu.VMEM((PCH,), jnp.int32),
        pltpu.VMEM((PCH,), jnp.int32),
        pltpu.VMEM((PCH,), jnp.int32),
        pltpu.VMEM((PCH,), jnp.int32),
        pltpu.VMEM((PCH,), jnp.int32),
        pltpu.VMEM((PCH,), jnp.float32),
        pltpu.VMEM((PCH,), jnp.float32),
        pltpu.VMEM((PCH,), jnp.float32),
        pltpu.VMEM((PCH,), jnp.float32),
        pltpu.VMEM((L,), jnp.float32),
        pltpu.SemaphoreType.DMA,
        pltpu.SemaphoreType.DMA,
        pltpu.SemaphoreType.DMA,
        pltpu.SemaphoreType.DMA,
        pltpu.SemaphoreType.DMA,
        pltpu.SemaphoreType.DMA,
        pltpu.SemaphoreType.DMA,
        pltpu.SemaphoreType.DMA,
    ],
)
def _k4_pairs(c_hbm, a_hbm, b_hbm, age_hbm,
              sh_cp, a0, a1, b0, b1, ma0, ma1, mb0, mb1,
              ca0, ca1, cb0, cb1, ostg,
              ia0, ia1, ib0, ib1, ga0, ga1, gb0, gb1):
    clo, chi, pkb = ca0, ca1, cb0
    sid = lax.axis_index("s")
    wid = _wid()
    half = N // 2
    wseg = half // NS

    def _stage(j, _):
        woff = sid * wseg + j * PCH
        pltpu.sync_copy(c_hbm.at[pl.ds(woff, PCH)], clo)
        pltpu.sync_copy(c_hbm.at[pl.ds(woff + half, PCH)], chi)

        def _pk(i, _):
            s = pl.ds(i * L, L)
            blo = lax.bitcast_convert_type(clo[s], jnp.int32)
            bhi = lax.bitcast_convert_type(chi[s], jnp.int32)
            rlo = lax.shift_right_logical(
                blo + 0x7FFF + (lax.shift_right_logical(blo, 16) & 1), 16)
            rhi = lax.shift_right_logical(
                bhi + 0x7FFF + (lax.shift_right_logical(bhi, 16) & 1), 16)
            pkb[s] = lax.bitcast_convert_type(rlo | lax.shift_left(rhi, 16),
                                              jnp.float32)
            return 0
        lax.fori_loop(0, PCH // L, _pk, 0, unroll=8)
        pltpu.sync_copy(pkb, sh_cp.at[pl.ds(woff, PCH)])
        return 0
    lax.fori_loop(0, wseg // PCH, _stage, 0)
    plsc.subcore_barrier()

    base = wid * PPT
    zf = jnp.zeros((L,), jnp.float32)
    abufs, bbufs = (a0, a1), (b0, b1)
    mabufs, mbbufs = (ma0, ma1), (mb0, mb1)
    cabufs, cbbufs = (ca0, ca1), (cb0, cb1)
    iasems, ibsems = (ia0, ia1), (ib0, ib1)
    gasems, gbsems = (ga0, ga1), (gb0, gb1)
    wmask = jnp.int32(half - 1)

    def _issue_idx(k, s):
        kk = lax.rem(k, jnp.int32(NCHUNK))
        off = base + kk * PCH
        pltpu.async_copy(a_hbm.at[pl.ds(off, PCH)], abufs[s], iasems[s])
        pltpu.async_copy(b_hbm.at[pl.ds(off, PCH)], bbufs[s], ibsems[s])

    def _wait_idx(s):
        pltpu.make_async_copy(a_hbm.at[pl.ds(0, PCH)], abufs[s], iasems[s]).wait()
        pltpu.make_async_copy(b_hbm.at[pl.ds(0, PCH)], bbufs[s], ibsems[s]).wait()

    def _mask_idx(s):
        def _m(i, _):
            sl = pl.ds(i * L, L)
            mabufs[s][sl] = abufs[s][sl] & wmask
            mbbufs[s][sl] = bbufs[s][sl] & wmask
            return 0
        lax.fori_loop(0, PCH // L, _m, 0, unroll=8)

    def _issue_gather(k, s):
        pltpu.async_copy(sh_cp.at[mabufs[s]], cabufs[s], gasems[s])
        pltpu.async_copy(sh_cp.at[mbbufs[s]], cbbufs[s], gbsems[s])

    def _wait_gather(s):
        pltpu.make_async_copy(sh_cp.at[mabufs[s]], cabufs[s], gasems[s]).wait()
        pltpu.make_async_copy(sh_cp.at[mbbufs[s]], cbbufs[s], gbsems[s]).wait()

    def _decode(wordf, idx):
        word = lax.bitcast_convert_type(wordf, jnp.int32)
        hi = lax.shift_right_logical(idx, 18) > 0
        bits = jnp.where(hi, word & jnp.int32(-65536), lax.shift_left(word, 16))
        return lax.bitcast_convert_type(bits, jnp.float32)

    def _compute(k, s, acc):
        del k

        def _vec(i, acc2):
            sl = pl.ds(i * L, L)
            va = _decode(cabufs[s][sl], abufs[s][sl])
            vb = _decode(cbbufs[s][sl], bbufs[s][sl])
            d = va - vb
            return acc2 + d * d
        return lax.fori_loop(0, PCH // L, _vec, acc, unroll=8)

    _issue_idx(jnp.int32(0), 0)
    _issue_idx(jnp.int32(1), 1)

    def _super(h, acc):
        k0 = h * 2
        _wait_idx(0)
        _mask_idx(0)
        _issue_gather(k0, 0)
        _wait_idx(1)
        _mask_idx(1)
        _issue_gather(k0 + 1, 1)
        _wait_gather(0)
        _issue_idx(k0 + 2, 0)
        acc = _compute(k0, 0, acc)
        _wait_gather(1)
        _issue_idx(k0 + 3, 1)
        acc = _compute(k0 + 1, 1, acc)
        return acc
    acc = lax.fori_loop(0, NCHUNK // 2, _super, zf)
    _wait_idx(0)
    _wait_idx(1)

    ostg[pl.ds(0, L)] = acc
    pltpu.sync_copy(ostg, age_hbm.at[pl.ds(wid * L, L)])


def kernel(mem_pred, mem_gt, a, b):
    c, hists = _k1_hist(mem_pred, mem_gt)
    mid = _k2_scan(hists)
    pad_p, rank_p, mse_p = _k3_dots(mem_pred, mem_gt, mid)
    age_p = _k4_pairs(c, a, b)
    inv_n = jnp.float32(1.0 / N)
    l_pad = jnp.sum(pad_p) * inv_n
    l_rank = jnp.sum(rank_p) * inv_n
    mse = jnp.sum(mse_p) * inv_n
    l_age = jnp.sum(age_p) * jnp.float32(1.0 / M)
    return 20.0 * (l_pad + l_rank + l_age) + mse
